# baseline (device time: 234443 ns/iter reference)
import jax
import jax.numpy as jnp
from jax import lax
from jax.experimental import pallas as pl
from jax.experimental.pallas import tpu as pltpu

N_TILES = 8
NC = 16


def kernel(x, W):
    t, d = x.shape
    _, v_loc = W.shape
    v_glob = 2 * v_loc
    tile = v_loc // N_TILES

    def body(
        x_ref, w_ref, out_ref, w_tiles, logits, sbuf, xrbuf, stat_tx,
        stat_rx, wsems, csem, cpsems, stat_sems, sems,
    ):
        mx = lax.axis_index("x")
        my = lax.axis_index("y")
        nbr = (1 - mx, my)
        nbr_y = (mx, 1 - my)
        half_rows = t // 2
        rc = half_rows // NC
        r0 = my * half_rows
        q0 = (1 - my) * half_rows
        own_c0 = mx * v_loc
        oth_c0 = (1 - mx) * v_loc

        barrier = pltpu.get_barrier_semaphore()
        for nb in (nbr, nbr_y):
            pl.semaphore_signal(
                barrier, inc=1, device_id=nb,
                device_id_type=pl.DeviceIdType.MESH,
            )
        pl.semaphore_wait(barrier, 2)

        xv = x_ref[...]

        def wdma(i, slot):
            return pltpu.make_async_copy(
                w_ref.at[:, pl.ds(i * tile, tile)],
                w_tiles.at[slot],
                wsems.at[slot],
            )

        wdma(0, 0).start()
        m_loc = jnp.full((t, 1), -1e30, jnp.float32)
        for i in range(N_TILES):
            slot = i % 2
            if i + 1 < N_TILES:
                wdma(i + 1, 1 - slot).start()
            wdma(i, slot).wait()
            tl = jnp.dot(xv, w_tiles[slot], preferred_element_type=jnp.float32)
            logits[:, i * tile : (i + 1) * tile] = tl
            m_loc = jnp.maximum(m_loc, jnp.max(tl, axis=1, keepdims=True))

        stage = pltpu.make_async_copy(
            logits.at[pl.ds(r0, half_rows), :], sbuf, csem
        )
        stage.start()
        stage.wait()
        x_out = []
        for c in range(NC):
            cs = slice(c * rc, (c + 1) * rc)
            rd = pltpu.make_async_remote_copy(
                src_ref=sbuf.at[cs, :],
                dst_ref=xrbuf.at[cs, :],
                send_sem=sems.at[0, c],
                recv_sem=sems.at[1, c],
                device_id=nbr,
                device_id_type=pl.DeviceIdType.MESH,
            )
            rd.start()
            x_out.append(rd)

        s_loc = jnp.zeros((t, 1), jnp.float32)
        for i in range(N_TILES):
            sl = slice(i * tile, (i + 1) * tile)
            e_t = jnp.exp(logits[:, sl] - m_loc)
            logits[:, sl] = e_t
            s_loc = s_loc + jnp.sum(e_t, axis=1, keepdims=True)

        stat_tx[:, 0:128] = jnp.broadcast_to(m_loc, (t, 128))
        stat_tx[:, 128:256] = jnp.broadcast_to(s_loc, (t, 128))
        stat_rdma = pltpu.make_async_remote_copy(
            src_ref=stat_tx,
            dst_ref=stat_rx,
            send_sem=stat_sems.at[0],
            recv_sem=stat_sems.at[1],
            device_id=nbr,
            device_id_type=pl.DeviceIdType.MESH,
        )
        stat_rdma.start()
        stat_rdma.wait()

        m_rem = stat_rx[:, 0:1]
        s_rem = stat_rx[:, 128:129]
        m_glob = jnp.maximum(m_loc, m_rem)
        s_glob = s_loc * jnp.exp(m_loc - m_glob) + s_rem * jnp.exp(
            m_rem - m_glob
        )
        inv_s = 1.0 / s_glob
        scale = jnp.exp(m_loc - m_glob) * inv_s

        for i in range(N_TILES):
            sl = slice(i * tile, (i + 1) * tile)
            logits[:, sl] = logits[:, sl] * scale

        local_cp = pltpu.make_async_copy(
            logits, out_ref.at[:, pl.ds(own_c0, v_loc)], cpsems.at[0]
        )
        local_cp.start()

        is_y0 = my == 0
        mg_x = jnp.where(is_y0, m_glob[0:half_rows, :], m_glob[half_rows:t, :])
        is_x = jnp.where(is_y0, inv_s[0:half_rows, :], inv_s[half_rows:t, :])

        y_out = []
        for c in range(NC):
            cs = slice(c * rc, (c + 1) * rc)
            x_in = pltpu.make_async_remote_copy(
                src_ref=sbuf.at[cs, :],
                dst_ref=xrbuf.at[cs, :],
                send_sem=sems.at[0, c],
                recv_sem=sems.at[1, c],
                device_id=nbr,
                device_id_type=pl.DeviceIdType.MESH,
            )
            x_in.wait_recv()
            xrbuf[cs, :] = jnp.exp(xrbuf[cs, :] - mg_x[cs, :]) * is_x[cs, :]
            fwd = pltpu.make_async_remote_copy(
                src_ref=xrbuf.at[cs, :],
                dst_ref=out_ref.at[
                    pl.ds(r0 + c * rc, rc), pl.ds(oth_c0, v_loc)
                ],
                send_sem=sems.at[2, c],
                recv_sem=sems.at[3, c],
                device_id=nbr_y,
                device_id_type=pl.DeviceIdType.MESH,
            )
            fwd.start()
            y_out.append(fwd)

        cp_x = pltpu.make_async_copy(
            xrbuf,
            out_ref.at[pl.ds(r0, half_rows), pl.ds(oth_c0, v_loc)],
            cpsems.at[1],
        )
        cp_x.start()

        for c in range(NC):
            y_in = pltpu.make_async_remote_copy(
                src_ref=xrbuf.at[slice(c * rc, (c + 1) * rc), :],
                dst_ref=out_ref.at[
                    pl.ds(q0 + c * rc, rc), pl.ds(oth_c0, v_loc)
                ],
                send_sem=sems.at[2, c],
                recv_sem=sems.at[3, c],
                device_id=nbr_y,
                device_id_type=pl.DeviceIdType.MESH,
            )
            y_in.wait_recv()
        for rd in x_out:
            rd.wait_send()
        for rd in y_out:
            rd.wait_send()
        local_cp.wait()
        cp_x.wait()

    return pl.pallas_call(
        body,
        out_shape=jax.ShapeDtypeStruct((t, v_glob), jnp.float32),
        in_specs=[
            pl.BlockSpec(memory_space=pltpu.VMEM),
            pl.BlockSpec(memory_space=pl.ANY),
        ],
        out_specs=pl.BlockSpec(memory_space=pl.ANY),
        scratch_shapes=[
            pltpu.VMEM((2, d, tile), jnp.float32),
            pltpu.VMEM((t, v_loc), jnp.float32),
            pltpu.VMEM((t // 2, v_loc), jnp.float32),
            pltpu.VMEM((t // 2, v_loc), jnp.float32),
            pltpu.VMEM((t, 256), jnp.float32),
            pltpu.VMEM((t, 256), jnp.float32),
            pltpu.SemaphoreType.DMA((2,)),
            pltpu.SemaphoreType.DMA,
            pltpu.SemaphoreType.DMA((2,)),
            pltpu.SemaphoreType.DMA((2,)),
            pltpu.SemaphoreType.DMA((4, NC)),
        ],
        compiler_params=pltpu.CompilerParams(
            collective_id=0,
            vmem_limit_bytes=62 * 1024 * 1024,
        ),
    )(x, W)


# device time: 150773 ns/iter; 1.5549x vs baseline; 1.5549x over previous
import jax
import jax.numpy as jnp
from jax import lax
from jax.experimental import pallas as pl
from jax.experimental.pallas import tpu as pltpu

N_TILES = 8
NC = 16


def kernel(x, W):
    t, d = x.shape
    _, v_loc = W.shape
    v_glob = 2 * v_loc
    tile = v_loc // N_TILES

    def body(
        x_ref, w_ref, out_ref, w_tiles, logits, sbuf, xrbuf, stat_tx,
        stat_rx, wsems, csem, cpsems, stat_sems, sems,
    ):
        mx = lax.axis_index("x")
        my = lax.axis_index("y")
        nbr = (1 - mx, my)
        nbr_y = (mx, 1 - my)
        half_rows = t // 2
        rc = half_rows // NC
        r0 = my * half_rows
        q0 = (1 - my) * half_rows
        own_c0 = mx * v_loc
        oth_c0 = (1 - mx) * v_loc

        barrier = pltpu.get_barrier_semaphore()
        for nb in (nbr, nbr_y):
            pl.semaphore_signal(
                barrier, inc=1, device_id=nb,
                device_id_type=pl.DeviceIdType.MESH,
            )
        pl.semaphore_wait(barrier, 2)

        xv = x_ref[...]

        def wdma(i, slot):
            return pltpu.make_async_copy(
                w_ref.at[:, pl.ds(i * tile, tile)],
                w_tiles.at[slot],
                wsems.at[slot],
            )

        wdma(0, 0).start()
        m_loc = jnp.full((t, 1), -1e30, jnp.float32)
        for i in range(N_TILES):
            slot = i % 2
            if i + 1 < N_TILES:
                wdma(i + 1, 1 - slot).start()
            wdma(i, slot).wait()
            tl = jnp.dot(xv, w_tiles[slot], preferred_element_type=jnp.float32)
            logits[:, i * tile : (i + 1) * tile] = tl
            m_loc = jnp.maximum(m_loc, jnp.max(tl, axis=1, keepdims=True))

        s_loc = jnp.zeros((t, 1), jnp.float32)
        for i in range(N_TILES):
            sl = slice(i * tile, (i + 1) * tile)
            e_t = jnp.exp(logits[:, sl] - m_loc)
            logits[:, sl] = e_t
            s_loc = s_loc + jnp.sum(e_t, axis=1, keepdims=True)

        stat_tx[:, 0:128] = jnp.broadcast_to(m_loc, (t, 128))
        stat_tx[:, 128:256] = jnp.broadcast_to(s_loc, (t, 128))
        stat_rdma = pltpu.make_async_remote_copy(
            src_ref=stat_tx,
            dst_ref=stat_rx,
            send_sem=stat_sems.at[0],
            recv_sem=stat_sems.at[1],
            device_id=nbr,
            device_id_type=pl.DeviceIdType.MESH,
        )
        stat_rdma.start()

        stage = pltpu.make_async_copy(
            logits.at[pl.ds(r0, half_rows), :], sbuf, csem
        )
        stage.start()
        stage.wait()
        x_out = []
        for c in range(NC):
            cs = slice(c * rc, (c + 1) * rc)
            rd = pltpu.make_async_remote_copy(
                src_ref=sbuf.at[cs, :],
                dst_ref=xrbuf.at[cs, :],
                send_sem=sems.at[0, c],
                recv_sem=sems.at[1, c],
                device_id=nbr,
                device_id_type=pl.DeviceIdType.MESH,
            )
            rd.start()
            x_out.append(rd)

        stat_rdma.wait()

        m_rem = stat_rx[:, 0:1]
        s_rem = stat_rx[:, 128:129]
        m_glob = jnp.maximum(m_loc, m_rem)
        s_glob = s_loc * jnp.exp(m_loc - m_glob) + s_rem * jnp.exp(
            m_rem - m_glob
        )
        inv_s = 1.0 / s_glob
        scale = jnp.exp(m_loc - m_glob) * inv_s

        for i in range(N_TILES):
            sl = slice(i * tile, (i + 1) * tile)
            logits[:, sl] = logits[:, sl] * scale

        local_cp = pltpu.make_async_copy(
            logits, out_ref.at[:, pl.ds(own_c0, v_loc)], cpsems.at[0]
        )
        local_cp.start()

        corr = jnp.exp(m_rem - m_glob) * inv_s
        is_y0 = my == 0
        corr_x = jnp.where(is_y0, corr[0:half_rows, :], corr[half_rows:t, :])

        y_out = []
        for c in range(NC):
            cs = slice(c * rc, (c + 1) * rc)
            x_in = pltpu.make_async_remote_copy(
                src_ref=sbuf.at[cs, :],
                dst_ref=xrbuf.at[cs, :],
                send_sem=sems.at[0, c],
                recv_sem=sems.at[1, c],
                device_id=nbr,
                device_id_type=pl.DeviceIdType.MESH,
            )
            x_in.wait_recv()
            xrbuf[cs, :] = xrbuf[cs, :] * corr_x[cs, :]
            fwd = pltpu.make_async_remote_copy(
                src_ref=xrbuf.at[cs, :],
                dst_ref=out_ref.at[
                    pl.ds(r0 + c * rc, rc), pl.ds(oth_c0, v_loc)
                ],
                send_sem=sems.at[2, c],
                recv_sem=sems.at[3, c],
                device_id=nbr_y,
                device_id_type=pl.DeviceIdType.MESH,
            )
            fwd.start()
            y_out.append(fwd)

        cp_x = pltpu.make_async_copy(
            xrbuf,
            out_ref.at[pl.ds(r0, half_rows), pl.ds(oth_c0, v_loc)],
            cpsems.at[1],
        )
        cp_x.start()

        for c in range(NC):
            y_in = pltpu.make_async_remote_copy(
                src_ref=xrbuf.at[slice(c * rc, (c + 1) * rc), :],
                dst_ref=out_ref.at[
                    pl.ds(q0 + c * rc, rc), pl.ds(oth_c0, v_loc)
                ],
                send_sem=sems.at[2, c],
                recv_sem=sems.at[3, c],
                device_id=nbr_y,
                device_id_type=pl.DeviceIdType.MESH,
            )
            y_in.wait_recv()
        for rd in x_out:
            rd.wait_send()
        for rd in y_out:
            rd.wait_send()
        local_cp.wait()
        cp_x.wait()

    return pl.pallas_call(
        body,
        out_shape=jax.ShapeDtypeStruct((t, v_glob), jnp.float32),
        in_specs=[
            pl.BlockSpec(memory_space=pltpu.VMEM),
            pl.BlockSpec(memory_space=pl.ANY),
        ],
        out_specs=pl.BlockSpec(memory_space=pl.ANY),
        scratch_shapes=[
            pltpu.VMEM((2, d, tile), jnp.float32),
            pltpu.VMEM((t, v_loc), jnp.float32),
            pltpu.VMEM((t // 2, v_loc), jnp.float32),
            pltpu.VMEM((t // 2, v_loc), jnp.float32),
            pltpu.VMEM((t, 256), jnp.float32),
            pltpu.VMEM((t, 256), jnp.float32),
            pltpu.SemaphoreType.DMA((2,)),
            pltpu.SemaphoreType.DMA,
            pltpu.SemaphoreType.DMA((2,)),
            pltpu.SemaphoreType.DMA((2,)),
            pltpu.SemaphoreType.DMA((4, NC)),
        ],
        compiler_params=pltpu.CompilerParams(
            collective_id=0,
            vmem_limit_bytes=62 * 1024 * 1024,
        ),
    )(x, W)
